# baseline (speedup 1.0000x reference)
import functools

import jax
import jax.numpy as jnp
from jax import lax
from jax.experimental import pallas as pl
from jax.experimental.pallas import tpu as pltpu
from jax.experimental.pallas import tpu_sc as plsc

_N = 10000
_E = 320000
_EPS = 1e-5
_NC, _NS = 2, 16
_NW = _NC * _NS
_CHUNK = 128
_K = 80
_KH = _K // 2
_EPW = _K * _CHUNK
_EPAD = _NW * _EPW
_NACC = 10112
_SLICE = _NACC // _NS
_DEGW = 16


def _sc_mesh():
    return plsc.VectorSubcoreMesh(core_axis_name="c", subcore_axis_name="s",
                                  num_cores=_NC, num_subcores=_NS)


@functools.cache
def _make_deg():
    return functools.partial(
        pl.kernel,
        out_type=jax.ShapeDtypeStruct((_NW, 128, 128), jnp.float32),
        mesh=_sc_mesh(),
        compiler_params=pltpu.CompilerParams(needs_layout_passes=False),
        scratch_types=[
            pltpu.VMEM((_K, _CHUNK), jnp.int32),
            pltpu.VMEM((128, 128), jnp.float32),
        ],
    )(_deg_body)


def _deg_body(col_hbm, out_hbm, col_v, hist):
    c = lax.axis_index("c")
    s = lax.axis_index("s")
    wid = c * _NS + s
    pltpu.sync_copy(col_hbm.at[wid], col_v)
    zeros16 = jnp.zeros((16,), jnp.float32)

    def zero(i, carry):
        hist[i // 8, pl.ds((i % 8) * 16, 16)] = zeros16
        return carry

    lax.fori_loop(0, 128 * 8, zero, 0)
    ones16 = jnp.ones((16,), jnp.float32)

    def step(i, carry):
        col16 = col_v[i // 8, pl.ds((i % 8) * 16, 16)]
        hi = lax.shift_right_logical(col16, 7)
        lo = lax.bitwise_and(col16, 127)
        plsc.addupdate_scatter(hist, [hi, lo], ones16)
        return carry

    lax.fori_loop(0, _K * 8, step, 0)
    pltpu.sync_copy(hist, out_hbm.at[wid])


@functools.cache
def _make_scatter(do):
    def _scatter(y_hbm, row_hbm, col_hbm, zero_hbm, out_hbm,
                 row_v, col_v, buf0, buf1, acc, sg0, sg1, ss0, ss1):
        c = lax.axis_index("c")
        s = lax.axis_index("s")
        wid = c * _NS + s
        pltpu.sync_copy(zero_hbm.at[pl.ds(s * _SLICE, _SLICE)],
                        acc.at[pl.ds(s * _SLICE, _SLICE)])

        for h in range(2):
            pltpu.sync_copy(row_hbm.at[wid, pl.ds(h * _KH, _KH)], row_v)
            pltpu.sync_copy(col_hbm.at[wid, pl.ds(h * _KH, _KH)], col_v)
            if h == 0:
                plsc.subcore_barrier()

            pltpu.async_copy(y_hbm.at[row_v.at[0]], buf0, sg0)
            pltpu.async_copy(y_hbm.at[row_v.at[1]], buf1, sg1)

            def pair(i, carry):
                j = 2 * i
                pltpu.make_async_copy(y_hbm.at[row_v.at[j]], buf0, sg0).wait()
                pltpu.async_copy(buf0, acc.at[col_v.at[j]], ss0, add=True)
                pltpu.make_async_copy(y_hbm.at[row_v.at[j + 1]], buf1,
                                      sg1).wait()
                pltpu.async_copy(buf1, acc.at[col_v.at[j + 1]], ss1, add=True)

                @pl.when(j + 2 < _KH)
                def _():
                    pltpu.make_async_copy(buf0, acc.at[col_v.at[j]],
                                          ss0).wait()
                    pltpu.async_copy(y_hbm.at[row_v.at[j + 2]], buf0, sg0)

                @pl.when(j + 3 < _KH)
                def _():
                    pltpu.make_async_copy(buf1, acc.at[col_v.at[j + 1]],
                                          ss1).wait()
                    pltpu.async_copy(y_hbm.at[row_v.at[j + 3]], buf1, sg1)

                return carry

            lax.fori_loop(0, _KH // 2, pair, 0)
            pltpu.make_async_copy(buf0, acc.at[col_v.at[0]], ss0).wait()
            pltpu.make_async_copy(buf1, acc.at[col_v.at[1]], ss1).wait()

        plsc.subcore_barrier()
        pltpu.sync_copy(acc.at[pl.ds(s * _SLICE, _SLICE)],
                        out_hbm.at[c, pl.ds(s * _SLICE, _SLICE)])

    return functools.partial(
        pl.kernel,
        out_type=jax.ShapeDtypeStruct((_NC, _NACC, do), jnp.float32),
        mesh=_sc_mesh(),
        scratch_types=[
            pltpu.VMEM((_KH, _CHUNK), jnp.int32),
            pltpu.VMEM((_KH, _CHUNK), jnp.int32),
            pltpu.VMEM((_CHUNK, do), jnp.float32),
            pltpu.VMEM((_CHUNK, do), jnp.float32),
            pltpu.VMEM_SHARED((_NACC, do), jnp.float32),
            pltpu.SemaphoreType.DMA,
            pltpu.SemaphoreType.DMA,
            pltpu.SemaphoreType.DMA,
            pltpu.SemaphoreType.DMA,
        ],
    )(_scatter)


_BLK = 2000
_NBLK = _N // _BLK
_F32 = jnp.float32


def _softmax_rows(t):
    t = t - jnp.max(t, axis=1, keepdims=True)
    et = jnp.exp(t)
    return et / jnp.sum(et, axis=1, keepdims=True)


def _bn_fold(rm, rv, g, b):
    a = g * lax.rsqrt(rv + _EPS)
    const = jnp.sum(b - rm * a, axis=0, keepdims=True)
    return a, const


def _tc_deg_body(degp_ref, dinv_ref):
    deg = jnp.sum(degp_ref[...], axis=0) + 1.0
    dinv_ref[...] = lax.rsqrt(deg)


def _tc_deg(degp):
    return pl.pallas_call(
        _tc_deg_body,
        out_shape=jax.ShapeDtypeStruct((128, 128), _F32),
    )(degp)


def _tc_prep_body(dinv_ref, x_ref, w_ref, y_ref):
    y_ref[...] = dinv_ref[...] * jnp.dot(x_ref[...], w_ref[...],
                                         preferred_element_type=_F32)


def _tc_prep(dinv, x, w0):
    return pl.pallas_call(
        _tc_prep_body,
        grid=(_NBLK,),
        in_specs=[
            pl.BlockSpec((_BLK, 1), lambda i: (i, 0)),
            pl.BlockSpec((_BLK, 128), lambda i: (i, 0)),
            pl.BlockSpec((128, 128), lambda i: (0, 0)),
        ],
        out_specs=pl.BlockSpec((_BLK, 128), lambda i: (i, 0)),
        out_shape=jax.ShapeDtypeStruct((_N, 128), _F32),
    )(dinv, x, w0)


def _tc_mid_body(p_ref, y_ref, dinv_ref, x_ref, r_ref, wn_ref,
                 rm_ref, rv_ref, g_ref, b_ref, xn_ref, yn_ref):
    dinv = dinv_ref[...]
    p = p_ref[...]
    xc = dinv * (p[0] + p[1] + y_ref[...])
    rm = rm_ref[...]
    t = lax.dot_general(xc, rm, (((1,), (1,)), ((), ())),
                        preferred_element_type=_F32)
    score = _softmax_rows(t)
    a, const = _bn_fold(rm, rv_ref[...], g_ref[...], b_ref[...])
    sa = lax.dot_general(score, a, (((1,), (0,)), ((), ())),
                         preferred_element_type=_F32)
    xbn = xc * sa + const
    xn = jnp.maximum(xbn, 0.0) + jnp.dot(x_ref[...], r_ref[...],
                                         preferred_element_type=_F32)
    xn_ref[...] = xn
    yn_ref[...] = dinv * jnp.dot(xn, wn_ref[...], preferred_element_type=_F32)


def _tc_mid(p, y, dinv, x, r, wn, rm, rv, g, b):
    dn = wn.shape[1]
    return pl.pallas_call(
        _tc_mid_body,
        grid=(_NBLK,),
        in_specs=[
            pl.BlockSpec((_NC, _BLK, 128), lambda i: (0, i, 0)),
            pl.BlockSpec((_BLK, 128), lambda i: (i, 0)),
            pl.BlockSpec((_BLK, 1), lambda i: (i, 0)),
            pl.BlockSpec((_BLK, 128), lambda i: (i, 0)),
            pl.BlockSpec((128, 128), lambda i: (0, 0)),
            pl.BlockSpec((128, dn), lambda i: (0, 0)),
            pl.BlockSpec((20, 128), lambda i: (0, 0)),
            pl.BlockSpec((20, 128), lambda i: (0, 0)),
            pl.BlockSpec((20, 128), lambda i: (0, 0)),
            pl.BlockSpec((20, 128), lambda i: (0, 0)),
        ],
        out_specs=[
            pl.BlockSpec((_BLK, 128), lambda i: (i, 0)),
            pl.BlockSpec((_BLK, dn), lambda i: (i, 0)),
        ],
        out_shape=[
            jax.ShapeDtypeStruct((_N, 128), _F32),
            jax.ShapeDtypeStruct((_N, dn), _F32),
        ],
    )(p, y, dinv, x, r, wn, rm, rv, g, b)


def _tc_fin_body(p_ref, y_ref, dinv_ref, x_ref, r_ref, w2_ref,
                 rm_ref, rv_ref, g_ref, b_ref,
                 frm_ref, frv_ref, fg_ref, fb_ref, xn_ref):
    dinv = dinv_ref[...]
    p = p_ref[...]
    agg = dinv * (p[0] + p[1] + y_ref[...])
    xc = jnp.dot(agg, w2_ref[...], preferred_element_type=_F32)
    rm = rm_ref[...]
    t = lax.dot_general(xc, rm, (((1,), (1,)), ((), ())),
                        preferred_element_type=_F32)
    score = _softmax_rows(t)
    a, const = _bn_fold(rm, rv_ref[...], g_ref[...], b_ref[...])
    sa = lax.dot_general(score, a, (((1,), (0,)), ((), ())),
                         preferred_element_type=_F32)
    xbn = xc * sa + const
    frm = frm_ref[...]
    t2 = lax.dot_general(xbn, frm, (((1,), (1,)), ((), ())),
                         preferred_element_type=_F32)
    score2 = _softmax_rows(t2)
    fa, fconst = _bn_fold(frm, frv_ref[...], fg_ref[...], fb_ref[...])
    sa2 = lax.dot_general(score2, fa, (((1,), (0,)), ((), ())),
                          preferred_element_type=_F32)
    xbn = xbn + 0.005 * (xbn * sa2 + fconst)
    xn_ref[...] = jnp.maximum(xbn, 0.0) + jnp.dot(
        x_ref[...], r_ref[...], preferred_element_type=_F32)


def _tc_fin(p, y, dinv, x, r, w2, rm, rv, g, b, frm, frv, fg, fb):
    return pl.pallas_call(
        _tc_fin_body,
        grid=(_NBLK,),
        in_specs=[
            pl.BlockSpec((_NC, _BLK, 128), lambda i: (0, i, 0)),
            pl.BlockSpec((_BLK, 128), lambda i: (i, 0)),
            pl.BlockSpec((_BLK, 1), lambda i: (i, 0)),
            pl.BlockSpec((_BLK, 128), lambda i: (i, 0)),
            pl.BlockSpec((128, 16), lambda i: (0, 0)),
            pl.BlockSpec((128, 16), lambda i: (0, 0)),
        ] + [pl.BlockSpec((20, 16), lambda i: (0, 0))] * 8,
        out_specs=pl.BlockSpec((_BLK, 16), lambda i: (i, 0)),
        out_shape=jax.ShapeDtypeStruct((_N, 16), _F32),
    )(p, y, dinv, x, r, w2, rm, rv, g, b, frm, frv, fg, fb)


def _pack_idx(v, pad_vals):
    vr = v.astype(jnp.int32).reshape(_NW, _E // _NW)
    pad = jnp.broadcast_to(pad_vals, (_NW, _EPW - _E // _NW))
    return jnp.concatenate([vr, pad], axis=1).reshape(_NW, _K, _CHUNK)


def kernel(x, edge_index, W0, W1, W2, R0, R1, R2,
           bn0_rm, bn0_rv, bn0_g, bn0_b,
           bn1_rm, bn1_rv, bn1_g, bn1_b,
           bn2_rm, bn2_rv, bn2_g, bn2_b,
           fin_rm, fin_rv, fin_g, fin_b):
    npad = _EPW - _E // _NW
    rowp = _pack_idx(edge_index[0], jnp.zeros((npad,), jnp.int32))
    colp = _pack_idx(edge_index[1],
                     _N + jnp.arange(npad, dtype=jnp.int32) % (_NACC - _N))
    z128 = jnp.zeros((_NACC, 128), _F32)

    degp = _make_deg()(colp)
    dinv = _tc_deg(degp).reshape(128 * 128, 1)[:_N]
    y0 = _tc_prep(dinv, x, W0)

    p0 = _make_scatter(128)(y0, rowp, colp, z128)
    x1, y1 = _tc_mid(p0, y0, dinv, x, R0, W1, bn0_rm, bn0_rv, bn0_g, bn0_b)

    p1 = _make_scatter(128)(y1, rowp, colp, z128)
    eye = jnp.eye(128, dtype=_F32)
    x2, y2 = _tc_mid(p1, y1, dinv, x1, R1, eye, bn1_rm, bn1_rv, bn1_g, bn1_b)

    p2 = _make_scatter(128)(y2, rowp, colp, z128)
    x3 = _tc_fin(p2, y2, dinv, x2, R2, W2, bn2_rm, bn2_rv, bn2_g, bn2_b,
                 fin_rm, fin_rv, fin_g, fin_b)

    return (x3, x, x1, x2, x3)


# speedup vs baseline: 1.0790x; 1.0790x over previous
import functools

import jax
import jax.numpy as jnp
from jax import lax
from jax.experimental import pallas as pl
from jax.experimental.pallas import tpu as pltpu
from jax.experimental.pallas import tpu_sc as plsc

_N = 10000
_E = 320000
_EPS = 1e-5
_NC, _NS = 2, 16
_NW = _NC * _NS
_CHUNK = 128
_K = 80
_KH = _K // 2
_EPW = _K * _CHUNK
_EPAD = _NW * _EPW
_NACC = 10112
_SLICE = _NACC // _NS
_DEGW = 16


def _sc_mesh():
    return plsc.VectorSubcoreMesh(core_axis_name="c", subcore_axis_name="s",
                                  num_cores=_NC, num_subcores=_NS)


@functools.cache
def _make_deg():
    return functools.partial(
        pl.kernel,
        out_type=jax.ShapeDtypeStruct((_NW, 128, 128), jnp.float32),
        mesh=_sc_mesh(),
        compiler_params=pltpu.CompilerParams(needs_layout_passes=False),
        scratch_types=[
            pltpu.VMEM((_K, _CHUNK), jnp.int32),
            pltpu.VMEM((128, 128), jnp.float32),
        ],
    )(_deg_body)


def _deg_body(col_hbm, out_hbm, col_v, hist):
    c = lax.axis_index("c")
    s = lax.axis_index("s")
    wid = c * _NS + s
    pltpu.sync_copy(col_hbm.at[wid], col_v)
    zeros16 = jnp.zeros((16,), jnp.float32)

    def zero(i, carry):
        hist[i // 8, pl.ds((i % 8) * 16, 16)] = zeros16
        return carry

    lax.fori_loop(0, 128 * 8, zero, 0)
    ones16 = jnp.ones((16,), jnp.float32)

    def step(i, carry):
        col16 = col_v[i // 8, pl.ds((i % 8) * 16, 16)]
        hi = lax.shift_right_logical(col16, 7)
        lo = lax.bitwise_and(col16, 127)
        plsc.addupdate_scatter(hist, [hi, lo], ones16)
        return carry

    lax.fori_loop(0, _K * 8, step, 0)
    pltpu.sync_copy(hist, out_hbm.at[wid])


@functools.cache
def _make_scatter(do):
    def _scatter(y_hbm, row_hbm, col_hbm, zero_hbm, out_hbm,
                 row_v, col_v, buf0, buf1, acc, sg0, sg1):
        c = lax.axis_index("c")
        s = lax.axis_index("s")
        wid = c * _NS + s
        pltpu.sync_copy(zero_hbm.at[pl.ds(s * _SLICE, _SLICE)],
                        acc.at[pl.ds(s * _SLICE, _SLICE)])

        for h in range(2):
            pltpu.sync_copy(row_hbm.at[wid, pl.ds(h * _KH, _KH)], row_v)
            pltpu.sync_copy(col_hbm.at[wid, pl.ds(h * _KH, _KH)], col_v)
            if h == 0:
                plsc.subcore_barrier()

            pltpu.async_copy(y_hbm.at[row_v.at[0]], buf0, sg0)

            def pair(i, carry):
                j = 2 * i
                pltpu.async_copy(y_hbm.at[row_v.at[j + 1]], buf1, sg1)
                pltpu.make_async_copy(y_hbm.at[row_v.at[j]], buf0, sg0).wait()
                pltpu.sync_copy(buf0, acc.at[col_v.at[j]], add=True)

                @pl.when(j + 2 < _KH)
                def _():
                    pltpu.async_copy(y_hbm.at[row_v.at[j + 2]], buf0, sg0)

                pltpu.make_async_copy(y_hbm.at[row_v.at[j + 1]], buf1,
                                      sg1).wait()
                pltpu.sync_copy(buf1, acc.at[col_v.at[j + 1]], add=True)
                return carry

            lax.fori_loop(0, _KH // 2, pair, 0)

        plsc.subcore_barrier()
        pltpu.sync_copy(acc.at[pl.ds(s * _SLICE, _SLICE)],
                        out_hbm.at[c, pl.ds(s * _SLICE, _SLICE)])

    return functools.partial(
        pl.kernel,
        out_type=jax.ShapeDtypeStruct((_NC, _NACC, do), jnp.float32),
        mesh=_sc_mesh(),
        scratch_types=[
            pltpu.VMEM((_KH, _CHUNK), jnp.int32),
            pltpu.VMEM((_KH, _CHUNK), jnp.int32),
            pltpu.VMEM((_CHUNK, do), jnp.float32),
            pltpu.VMEM((_CHUNK, do), jnp.float32),
            pltpu.VMEM_SHARED((_NACC, do), jnp.float32),
            pltpu.SemaphoreType.DMA,
            pltpu.SemaphoreType.DMA,
        ],
    )(_scatter)


_BLK = 2000
_NBLK = _N // _BLK
_F32 = jnp.float32


def _softmax_rows(t):
    t = t - jnp.max(t, axis=1, keepdims=True)
    et = jnp.exp(t)
    return et / jnp.sum(et, axis=1, keepdims=True)


def _bn_fold(rm, rv, g, b):
    a = g * lax.rsqrt(rv + _EPS)
    const = jnp.sum(b - rm * a, axis=0, keepdims=True)
    return a, const


def _tc_deg_body(degp_ref, dinv_ref):
    deg = jnp.sum(degp_ref[...], axis=0) + 1.0
    dinv_ref[...] = lax.rsqrt(deg)


def _tc_deg(degp):
    return pl.pallas_call(
        _tc_deg_body,
        out_shape=jax.ShapeDtypeStruct((128, 128), _F32),
    )(degp)


def _tc_prep_body(dinv_ref, x_ref, w_ref, y_ref):
    y_ref[...] = dinv_ref[...] * jnp.dot(x_ref[...], w_ref[...],
                                         preferred_element_type=_F32)


def _tc_prep(dinv, x, w0):
    return pl.pallas_call(
        _tc_prep_body,
        grid=(_NBLK,),
        in_specs=[
            pl.BlockSpec((_BLK, 1), lambda i: (i, 0)),
            pl.BlockSpec((_BLK, 128), lambda i: (i, 0)),
            pl.BlockSpec((128, 128), lambda i: (0, 0)),
        ],
        out_specs=pl.BlockSpec((_BLK, 128), lambda i: (i, 0)),
        out_shape=jax.ShapeDtypeStruct((_N, 128), _F32),
    )(dinv, x, w0)


def _tc_mid_body(p_ref, y_ref, dinv_ref, x_ref, r_ref, wn_ref,
                 rm_ref, rv_ref, g_ref, b_ref, xn_ref, yn_ref):
    dinv = dinv_ref[...]
    p = p_ref[...]
    xc = dinv * (p[0] + p[1] + y_ref[...])
    rm = rm_ref[...]
    t = lax.dot_general(xc, rm, (((1,), (1,)), ((), ())),
                        preferred_element_type=_F32)
    score = _softmax_rows(t)
    a, const = _bn_fold(rm, rv_ref[...], g_ref[...], b_ref[...])
    sa = lax.dot_general(score, a, (((1,), (0,)), ((), ())),
                         preferred_element_type=_F32)
    xbn = xc * sa + const
    xn = jnp.maximum(xbn, 0.0) + jnp.dot(x_ref[...], r_ref[...],
                                         preferred_element_type=_F32)
    xn_ref[...] = xn
    yn_ref[...] = dinv * jnp.dot(xn, wn_ref[...], preferred_element_type=_F32)


def _tc_mid(p, y, dinv, x, r, wn, rm, rv, g, b):
    dn = wn.shape[1]
    return pl.pallas_call(
        _tc_mid_body,
        grid=(_NBLK,),
        in_specs=[
            pl.BlockSpec((_NC, _BLK, 128), lambda i: (0, i, 0)),
            pl.BlockSpec((_BLK, 128), lambda i: (i, 0)),
            pl.BlockSpec((_BLK, 1), lambda i: (i, 0)),
            pl.BlockSpec((_BLK, 128), lambda i: (i, 0)),
            pl.BlockSpec((128, 128), lambda i: (0, 0)),
            pl.BlockSpec((128, dn), lambda i: (0, 0)),
            pl.BlockSpec((20, 128), lambda i: (0, 0)),
            pl.BlockSpec((20, 128), lambda i: (0, 0)),
            pl.BlockSpec((20, 128), lambda i: (0, 0)),
            pl.BlockSpec((20, 128), lambda i: (0, 0)),
        ],
        out_specs=[
            pl.BlockSpec((_BLK, 128), lambda i: (i, 0)),
            pl.BlockSpec((_BLK, dn), lambda i: (i, 0)),
        ],
        out_shape=[
            jax.ShapeDtypeStruct((_N, 128), _F32),
            jax.ShapeDtypeStruct((_N, dn), _F32),
        ],
    )(p, y, dinv, x, r, wn, rm, rv, g, b)


def _tc_fin_body(p_ref, y_ref, dinv_ref, x_ref, r_ref, w2_ref,
                 rm_ref, rv_ref, g_ref, b_ref,
                 frm_ref, frv_ref, fg_ref, fb_ref, xn_ref):
    dinv = dinv_ref[...]
    p = p_ref[...]
    agg = dinv * (p[0] + p[1] + y_ref[...])
    xc = jnp.dot(agg, w2_ref[...], preferred_element_type=_F32)
    rm = rm_ref[...]
    t = lax.dot_general(xc, rm, (((1,), (1,)), ((), ())),
                        preferred_element_type=_F32)
    score = _softmax_rows(t)
    a, const = _bn_fold(rm, rv_ref[...], g_ref[...], b_ref[...])
    sa = lax.dot_general(score, a, (((1,), (0,)), ((), ())),
                         preferred_element_type=_F32)
    xbn = xc * sa + const
    frm = frm_ref[...]
    t2 = lax.dot_general(xbn, frm, (((1,), (1,)), ((), ())),
                         preferred_element_type=_F32)
    score2 = _softmax_rows(t2)
    fa, fconst = _bn_fold(frm, frv_ref[...], fg_ref[...], fb_ref[...])
    sa2 = lax.dot_general(score2, fa, (((1,), (0,)), ((), ())),
                          preferred_element_type=_F32)
    xbn = xbn + 0.005 * (xbn * sa2 + fconst)
    xn_ref[...] = jnp.maximum(xbn, 0.0) + jnp.dot(
        x_ref[...], r_ref[...], preferred_element_type=_F32)


def _tc_fin(p, y, dinv, x, r, w2, rm, rv, g, b, frm, frv, fg, fb):
    return pl.pallas_call(
        _tc_fin_body,
        grid=(_NBLK,),
        in_specs=[
            pl.BlockSpec((_NC, _BLK, 128), lambda i: (0, i, 0)),
            pl.BlockSpec((_BLK, 128), lambda i: (i, 0)),
            pl.BlockSpec((_BLK, 1), lambda i: (i, 0)),
            pl.BlockSpec((_BLK, 128), lambda i: (i, 0)),
            pl.BlockSpec((128, 16), lambda i: (0, 0)),
            pl.BlockSpec((128, 16), lambda i: (0, 0)),
        ] + [pl.BlockSpec((20, 16), lambda i: (0, 0))] * 8,
        out_specs=pl.BlockSpec((_BLK, 16), lambda i: (i, 0)),
        out_shape=jax.ShapeDtypeStruct((_N, 16), _F32),
    )(p, y, dinv, x, r, w2, rm, rv, g, b, frm, frv, fg, fb)


def _pack_idx(v, pad_vals):
    vr = v.astype(jnp.int32).reshape(_NW, _E // _NW)
    pad = jnp.broadcast_to(pad_vals, (_NW, _EPW - _E // _NW))
    return jnp.concatenate([vr, pad], axis=1).reshape(_NW, _K, _CHUNK)


def kernel(x, edge_index, W0, W1, W2, R0, R1, R2,
           bn0_rm, bn0_rv, bn0_g, bn0_b,
           bn1_rm, bn1_rv, bn1_g, bn1_b,
           bn2_rm, bn2_rv, bn2_g, bn2_b,
           fin_rm, fin_rv, fin_g, fin_b):
    npad = _EPW - _E // _NW
    rowp = _pack_idx(edge_index[0], jnp.zeros((npad,), jnp.int32))
    colp = _pack_idx(edge_index[1],
                     _N + jnp.arange(npad, dtype=jnp.int32) % (_NACC - _N))
    z128 = jnp.zeros((_NACC, 128), _F32)

    degp = _make_deg()(colp)
    dinv = _tc_deg(degp).reshape(128 * 128, 1)[:_N]
    y0 = _tc_prep(dinv, x, W0)

    p0 = _make_scatter(128)(y0, rowp, colp, z128)
    x1, y1 = _tc_mid(p0, y0, dinv, x, R0, W1, bn0_rm, bn0_rv, bn0_g, bn0_b)

    p1 = _make_scatter(128)(y1, rowp, colp, z128)
    eye = jnp.eye(128, dtype=_F32)
    x2, y2 = _tc_mid(p1, y1, dinv, x1, R1, eye, bn1_rm, bn1_rv, bn1_g, bn1_b)

    p2 = _make_scatter(128)(y2, rowp, colp, z128)
    x3 = _tc_fin(p2, y2, dinv, x2, R2, W2, bn2_rm, bn2_rv, bn2_g, bn2_b,
                 fin_rm, fin_rv, fin_g, fin_b)

    return (x3, x, x1, x2, x3)
